# baseline (speedup 1.0000x reference)
import functools

import jax
import jax.numpy as jnp
from jax.experimental import pallas as pl


def _elu(v):
    return jnp.where(v > 0, v, jnp.exp(jnp.minimum(v, 0.0)) - 1.0)


def _mm_kernel(x_ref, w_ref, o_ref):
    o_ref[...] = jnp.dot(
        x_ref[...].astype(jnp.bfloat16),
        w_ref[...].astype(jnp.bfloat16),
        preferred_element_type=jnp.float32,
    ).astype(jnp.bfloat16)


def _agg_kernel(adj_ref, s_ref, b_ref, o_ref):
    acc = jnp.dot(
        adj_ref[...].astype(jnp.bfloat16),
        s_ref[...],
        preferred_element_type=jnp.float32,
    )
    o_ref[...] = _elu(acc + b_ref[...])


def _agg_mm_kernel(adj_ref, s_ref, b_ref, w_ref, o_ref):
    acc = jnp.dot(
        adj_ref[...].astype(jnp.bfloat16),
        s_ref[...],
        preferred_element_type=jnp.float32,
    )
    h = _elu(acc + b_ref[...])
    o_ref[...] = jnp.dot(
        h.astype(jnp.bfloat16),
        w_ref[...].astype(jnp.bfloat16),
        preferred_element_type=jnp.float32,
    ).astype(jnp.bfloat16)


@functools.partial(jax.jit, static_argnames=("block_m",))
def _forward(x, adj, W1, b1, W2, b2, block_m=400):
    n, nfeat = x.shape
    nhid = W1.shape[1]
    grid = (n // block_m,)

    support1 = pl.pallas_call(
        _mm_kernel,
        out_shape=jax.ShapeDtypeStruct((n, nhid), jnp.bfloat16),
    )(x, W1)

    b1r = b1.reshape(1, nhid)
    b2r = b2.reshape(1, nhid)

    adj_spec = pl.BlockSpec((block_m, n), lambda i: (i, 0))
    s_spec = pl.BlockSpec((n, nhid), lambda i: (0, 0))
    b_spec = pl.BlockSpec((1, nhid), lambda i: (0, 0))
    w_spec = pl.BlockSpec((nhid, nhid), lambda i: (0, 0))
    out_spec = pl.BlockSpec((block_m, nhid), lambda i: (i, 0))

    support2 = pl.pallas_call(
        _agg_mm_kernel,
        grid=grid,
        in_specs=[adj_spec, s_spec, b_spec, w_spec],
        out_specs=out_spec,
        out_shape=jax.ShapeDtypeStruct((n, nhid), jnp.bfloat16),
    )(adj, support1, b1r, W2)

    out = pl.pallas_call(
        _agg_kernel,
        grid=grid,
        in_specs=[adj_spec, s_spec, b_spec],
        out_specs=out_spec,
        out_shape=jax.ShapeDtypeStruct((n, nhid), jnp.float32),
    )(adj, support2, b2r)

    return out


def kernel(x, adj, W1, b1, W2, b2):
    return _forward(x, adj, W1, b1, W2, b2)


# speedup vs baseline: 1.0269x; 1.0269x over previous
import functools

import jax
import jax.numpy as jnp
from jax.experimental import pallas as pl
from jax.experimental.pallas import tpu as pltpu


def _elu(v):
    return jnp.where(v > 0, v, jnp.exp(jnp.minimum(v, 0.0)) - 1.0)


def _bf16(v):
    return v.astype(jnp.bfloat16)


def _make_fused_kernel(block_m):
    def fused_kernel(adj_ref, x_ref, w1_ref, b1_ref, w2_ref, b2_ref, o_ref,
                     sa_ref, sb_ref):
        layer = pl.program_id(0)
        i = pl.program_id(1)

        @pl.when((layer == 0) & (i == 0))
        def _():
            sa_ref[...] = _bf16(
                jnp.dot(_bf16(x_ref[...]), _bf16(w1_ref[...]),
                        preferred_element_type=jnp.float32))

        a = _bf16(adj_ref[...])

        @pl.when(layer == 0)
        def _():
            acc = jnp.dot(a, sa_ref[...], preferred_element_type=jnp.float32)
            h = _elu(acc + b1_ref[...])
            sb_ref[pl.ds(i * block_m, block_m), :] = _bf16(
                jnp.dot(_bf16(h), _bf16(w2_ref[...]),
                        preferred_element_type=jnp.float32))
            o_ref[...] = h

        @pl.when(layer == 1)
        def _():
            acc = jnp.dot(a, sb_ref[...], preferred_element_type=jnp.float32)
            o_ref[...] = _elu(acc + b2_ref[...])

    return fused_kernel


@functools.partial(jax.jit, static_argnames=("block_m",))
def _forward(x, adj, W1, b1, W2, b2, block_m=400):
    n, _ = x.shape
    nhid = W1.shape[1]

    return pl.pallas_call(
        _make_fused_kernel(block_m),
        grid=(2, n // block_m),
        in_specs=[
            pl.BlockSpec((block_m, n), lambda l, i: (i, 0)),
            pl.BlockSpec(x.shape, lambda l, i: (0, 0)),
            pl.BlockSpec(W1.shape, lambda l, i: (0, 0)),
            pl.BlockSpec((1, nhid), lambda l, i: (0, 0)),
            pl.BlockSpec(W2.shape, lambda l, i: (0, 0)),
            pl.BlockSpec((1, nhid), lambda l, i: (0, 0)),
        ],
        out_specs=pl.BlockSpec((block_m, nhid), lambda l, i: (i, 0)),
        out_shape=jax.ShapeDtypeStruct((n, nhid), jnp.float32),
        scratch_shapes=[
            pltpu.VMEM((n, nhid), jnp.bfloat16),
            pltpu.VMEM((n, nhid), jnp.bfloat16),
        ],
    )(adj, x, W1, b1.reshape(1, nhid), W2, b2.reshape(1, nhid))


def kernel(x, adj, W1, b1, W2, b2):
    return _forward(x, adj, W1, b1, W2, b2)


# speedup vs baseline: 1.0292x; 1.0023x over previous
import functools

import jax
import jax.numpy as jnp
from jax.experimental import pallas as pl
from jax.experimental.pallas import tpu as pltpu


def _elu(v):
    return jnp.where(v > 0, v, jnp.exp(jnp.minimum(v, 0.0)) - 1.0)


def _bf16(v):
    return v.astype(jnp.bfloat16)


def _make_fused_kernel(block_m):
    def fused_kernel(adj_ref, x_ref, w1_ref, b1_ref, w2_ref, b2_ref, o_ref,
                     sa_ref, sb_ref):
        layer = pl.program_id(0)
        i = pl.program_id(1)

        @pl.when((layer == 0) & (i == 0))
        def _():
            sa_ref[...] = _bf16(
                jnp.dot(_bf16(x_ref[...]), _bf16(w1_ref[...]),
                        preferred_element_type=jnp.float32))

        a = _bf16(adj_ref[...])

        @pl.when(layer == 0)
        def _():
            acc = jnp.dot(a, sa_ref[...], preferred_element_type=jnp.float32)
            h = _elu(acc + b1_ref[...])
            sb_ref[pl.ds(i * block_m, block_m), :] = _bf16(
                jnp.dot(_bf16(h), _bf16(w2_ref[...]),
                        preferred_element_type=jnp.float32))

        @pl.when(layer == 1)
        def _():
            acc = jnp.dot(a, sb_ref[...], preferred_element_type=jnp.float32)
            o_ref[...] = _elu(acc + b2_ref[...])

    return fused_kernel


@functools.partial(jax.jit, static_argnames=("block_m",))
def _forward(x, adj, W1, b1, W2, b2, block_m=400):
    n, _ = x.shape
    nhid = W1.shape[1]

    return pl.pallas_call(
        _make_fused_kernel(block_m),
        grid=(2, n // block_m),
        in_specs=[
            pl.BlockSpec((block_m, n), lambda l, i: (i, 0)),
            pl.BlockSpec(x.shape, lambda l, i: (0, 0)),
            pl.BlockSpec(W1.shape, lambda l, i: (0, 0)),
            pl.BlockSpec((1, nhid), lambda l, i: (0, 0)),
            pl.BlockSpec(W2.shape, lambda l, i: (0, 0)),
            pl.BlockSpec((1, nhid), lambda l, i: (0, 0)),
        ],
        out_specs=pl.BlockSpec((block_m, nhid), lambda l, i: (l * i, 0)),
        out_shape=jax.ShapeDtypeStruct((n, nhid), jnp.float32),
        scratch_shapes=[
            pltpu.VMEM((n, nhid), jnp.bfloat16),
            pltpu.VMEM((n, nhid), jnp.bfloat16),
        ],
    )(adj, x, W1, b1.reshape(1, nhid), W2, b2.reshape(1, nhid))


def kernel(x, adj, W1, b1, W2, b2):
    return _forward(x, adj, W1, b1, W2, b2)
